# baseline (speedup 1.0000x reference)
import functools

import jax
import jax.numpy as jnp
from jax import lax
from jax.experimental import pallas as pl
from jax.experimental.pallas import tpu as pltpu
from jax.experimental.pallas import tpu_sc as plsc


def _prim_kernel(d1_ref, d2_ref, rows1_ref, rows2_ref, md1_ref, md2_ref,
                 matched_ref):
    n, S, L = d1_ref.shape
    iota = (lax.broadcasted_iota(jnp.int32, (S, L), 0) * L
            + lax.broadcasted_iota(jnp.int32, (S, L), 1))
    INF = jnp.float32(jnp.inf)
    root = iota == 0

    def step(d_ref, masked, md, par):
        notin = masked < INF
        m = jnp.min(masked)
        v = jnp.min(jnp.where(masked == m, iota, jnp.int32(1 << 30)))
        row = d_ref[v]
        is_v = iota == v
        better = (row < masked) & notin
        masked_n = jnp.where(is_v, INF, jnp.where(better, row, masked))
        upd = better & jnp.logical_not(is_v)
        par_n = jnp.where(upd, v, par)
        md_n = jnp.where(upd, row, md)
        return masked_n, md_n, par_n

    def body(i, st):
        ma1, md1, par1, ma2, md2, par2 = st
        ma1, md1, par1 = step(d1_ref, ma1, md1, par1)
        ma2, md2, par2 = step(d2_ref, ma2, md2, par2)
        return (ma1, md1, par1, ma2, md2, par2)

    zero_i = jnp.zeros((S, L), jnp.int32)
    row0_1 = d1_ref[0]
    row0_2 = d2_ref[0]
    init = (jnp.where(root, INF, row0_1), row0_1, zero_i,
            jnp.where(root, INF, row0_2), row0_2, zero_i)
    _, md1, par1, _, md2, par2 = lax.fori_loop(0, n - 1, body, init)

    nonroot = jnp.logical_not(root)
    matched = jnp.sum(jnp.where((par1 == par2) & nonroot, 1.0, 0.0))

    rows1_ref[...] = par1 * (n // 128) + (iota >> 7)
    rows2_ref[...] = par2 * (n // 128) + (iota >> 7)
    md1_ref[...] = md1
    md2_ref[...] = md2
    matched_ref[...] = matched.reshape(1, 1)


def _finish_kernel(md1_ref, md2_ref, r12_ref, r21_ref, dist_ref, d12_ref,
                   d21_ref):
    S, L = md1_ref.shape
    n = S * L
    iota = (lax.broadcasted_iota(jnp.int32, (S, L), 0) * L
            + lax.broadcasted_iota(jnp.int32, (S, L), 1))
    sub8 = lax.broadcasted_iota(jnp.int32, (8, 128), 0)
    lane128 = lax.broadcasted_iota(jnp.int32, (8, 128), 1)
    nonroot = iota != 0

    def diag_select(r_ref):
        y = jnp.zeros((S, L), jnp.float32)
        for a in range(n // 128):
            p = jnp.zeros((8, 128), jnp.float32)
            for k in range(16):
                t = r_ref[a * 128 + 8 * k: a * 128 + 8 * k + 8, :]
                p = p + jnp.where(lane128 == 8 * k + sub8, t, 0.0)
            d = jnp.sum(p, axis=0, keepdims=True)
            y = y + jnp.where(sub8 == a, d, 0.0)
        return y

    s12 = diag_select(r12_ref)
    s21 = diag_select(r21_ref)
    e12 = jnp.where(nonroot, md1_ref[...] - s12, 0.0)
    e21 = jnp.where(nonroot, md2_ref[...] - s21, 0.0)
    d12 = jnp.sqrt(jnp.sum(e12 * e12))
    d21 = jnp.sqrt(jnp.sum(e21 * e21))
    dist_ref[...] = (d12 + d21).reshape(1, 1)
    d12_ref[...] = d12.reshape(1, 1)
    d21_ref[...] = d21.reshape(1, 1)


def _sc_info():
    try:
        info = plsc.get_sparse_core_info()
        return info.num_cores, info.num_subcores, info.num_lanes
    except Exception:
        return 2, 16, 16


def _make_sc_gather(n, nc, ns, lanes):
    nw = nc * ns
    bpw = n // nw
    halves = bpw // lanes
    mesh = plsc.VectorSubcoreMesh(
        core_axis_name="c", subcore_axis_name="s",
        num_cores=nc, num_subcores=ns)

    def body(d2f_ref, d1f_ref, rows1_ref, rows2_ref, r12_ref, r21_ref,
             idx_a, idx_b, rows_a, rows_b, sem_a, sem_b):
        wid = lax.axis_index("s") * nc + lax.axis_index("c")
        base = wid * bpw
        pltpu.sync_copy(rows1_ref.at[pl.ds(base, bpw)], idx_a)
        pltpu.sync_copy(rows2_ref.at[pl.ds(base, bpw)], idx_b)
        cp_a = pltpu.async_copy(d2f_ref.at[idx_a], rows_a, sem_a)
        cp_b = pltpu.async_copy(d1f_ref.at[idx_b], rows_b, sem_b)
        cp_a.wait()
        pltpu.sync_copy(rows_a, r12_ref.at[pl.ds(base, bpw)])
        cp_b.wait()
        pltpu.sync_copy(rows_b, r21_ref.at[pl.ds(base, bpw)])

    return pl.kernel(
        body,
        out_type=[jax.ShapeDtypeStruct((n, 128), jnp.float32)] * 2,
        mesh=mesh,
        scratch_types=[
            pltpu.VMEM((bpw,), jnp.int32),
            pltpu.VMEM((bpw,), jnp.int32),
            pltpu.VMEM((bpw, 128), jnp.float32),
            pltpu.VMEM((bpw, 128), jnp.float32),
            pltpu.SemaphoreType.DMA,
            pltpu.SemaphoreType.DMA,
        ],
    )


def kernel(distances1, distances2):
    n = distances1.shape[0]
    S = n // 128
    d1r = distances1.reshape(n, S, 128)
    d2r = distances2.reshape(n, S, 128)

    prim_out = [
        jax.ShapeDtypeStruct((S, 128), jnp.int32),
        jax.ShapeDtypeStruct((S, 128), jnp.int32),
        jax.ShapeDtypeStruct((S, 128), jnp.float32),
        jax.ShapeDtypeStruct((S, 128), jnp.float32),
        jax.ShapeDtypeStruct((1, 1), jnp.float32),
    ]
    rows1, rows2, md1, md2, matched = pl.pallas_call(
        _prim_kernel, out_shape=prim_out)(d1r, d2r)

    nc, ns, lanes = _sc_info()
    sc_gather = _make_sc_gather(n, nc, ns, lanes)
    r12, r21 = sc_gather(
        distances2.reshape(n * n // 128, 128),
        distances1.reshape(n * n // 128, 128),
        rows1.reshape(n), rows2.reshape(n))

    fin_out = [jax.ShapeDtypeStruct((1, 1), jnp.float32)] * 3
    dist, d12, d21 = pl.pallas_call(_finish_kernel, out_shape=fin_out)(
        md1, md2, r12, r21)
    return (dist[0, 0], matched[0, 0], d12[0, 0], d21[0, 0])


# speedup vs baseline: 1.9138x; 1.9138x over previous
import functools

import jax
import jax.numpy as jnp
from jax import lax
from jax.experimental import pallas as pl
from jax.experimental.pallas import tpu as pltpu
from jax.experimental.pallas import tpu_sc as plsc


def _prim_one_kernel(d_ref0, rows_ref, md_ref, par_ref):
    _, n, S, L = d_ref0.shape
    iota = (lax.broadcasted_iota(jnp.int32, (S, L), 0) * L
            + lax.broadcasted_iota(jnp.int32, (S, L), 1))
    INF = jnp.float32(jnp.inf)
    root = iota == 0
    sub_iota = lax.broadcasted_iota(jnp.int32, (S, 1), 0)

    def body(i, st):
        masked, md, par = st
        notin = masked < INF
        lane_idx = jnp.argmin(masked, axis=1).astype(jnp.int32).reshape(S, 1)
        lane_min = jnp.min(masked, axis=1).reshape(S, 1)
        lin = sub_iota * L + lane_idx
        lm = lane_min
        for sh in (4, 2, 1):
            lm_r = jnp.roll(lm, sh, axis=0)
            lin_r = jnp.roll(lin, sh, axis=0)
            take = (lm_r < lm) | ((lm_r == lm) & (lin_r < lin))
            lm = jnp.where(take, lm_r, lm)
            lin = jnp.where(take, lin_r, lin)
        v = lin[0, 0]
        row = d_ref0[0, v]
        is_v = iota == v
        better = (row < masked) & notin
        masked_n = jnp.where(is_v, INF, jnp.where(better, row, masked))
        upd = better & jnp.logical_not(is_v)
        par_n = jnp.where(upd, v, par)
        md_n = jnp.where(upd, row, md)
        return masked_n, md_n, par_n

    row0 = d_ref0[0, 0]
    init = (jnp.where(root, INF, row0), row0, jnp.zeros((S, L), jnp.int32))
    _, md, par = lax.fori_loop(0, n - 1, body, init)

    rows_ref[...] = (par * (n // 128) + (iota >> 7)).reshape(1, S, L)
    md_ref[...] = md.reshape(1, S, L)
    par_ref[...] = par.reshape(1, S, L)


def _finish_kernel(md_ref, par_ref, r12_ref, r21_ref, dist_ref, matched_ref,
                   d12_ref, d21_ref):
    _, S, L = md_ref.shape
    n = S * L
    iota = (lax.broadcasted_iota(jnp.int32, (S, L), 0) * L
            + lax.broadcasted_iota(jnp.int32, (S, L), 1))
    sub8 = lax.broadcasted_iota(jnp.int32, (8, 128), 0)
    lane128 = lax.broadcasted_iota(jnp.int32, (8, 128), 1)
    nonroot = iota != 0

    def diag_select(r_ref):
        y = jnp.zeros((S, L), jnp.float32)
        for a in range(n // 128):
            p = jnp.zeros((8, 128), jnp.float32)
            for k in range(16):
                t = r_ref[a * 128 + 8 * k: a * 128 + 8 * k + 8, :]
                p = p + jnp.where(lane128 == 8 * k + sub8, t, 0.0)
            d = jnp.sum(p, axis=0, keepdims=True)
            y = y + jnp.where(sub8 == a, d, 0.0)
        return y

    s12 = diag_select(r12_ref)
    s21 = diag_select(r21_ref)
    e12 = jnp.where(nonroot, md_ref[0] - s12, 0.0)
    e21 = jnp.where(nonroot, md_ref[1] - s21, 0.0)
    d12 = jnp.sqrt(jnp.sum(e12 * e12))
    d21 = jnp.sqrt(jnp.sum(e21 * e21))
    matched = jnp.sum(
        jnp.where((par_ref[0] == par_ref[1]) & nonroot, 1.0, 0.0))
    dist_ref[...] = (d12 + d21).reshape(1, 1)
    matched_ref[...] = matched.reshape(1, 1)
    d12_ref[...] = d12.reshape(1, 1)
    d21_ref[...] = d21.reshape(1, 1)


def _sc_info():
    try:
        info = plsc.get_sparse_core_info()
        return info.num_cores, info.num_subcores, info.num_lanes
    except Exception:
        return 2, 16, 16


def _make_sc_gather(n, nc, ns, lanes):
    nw = nc * ns
    bpw = n // nw
    halves = bpw // lanes
    mesh = plsc.VectorSubcoreMesh(
        core_axis_name="c", subcore_axis_name="s",
        num_cores=nc, num_subcores=ns)

    def body(d2f_ref, d1f_ref, rows1_ref, rows2_ref, r12_ref, r21_ref,
             idx_a, idx_b, rows_a, rows_b, sem_a, sem_b):
        wid = lax.axis_index("s") * nc + lax.axis_index("c")
        base = wid * bpw
        pltpu.sync_copy(rows1_ref.at[pl.ds(base, bpw)], idx_a)
        pltpu.sync_copy(rows2_ref.at[pl.ds(base, bpw)], idx_b)
        cp_a = pltpu.async_copy(d2f_ref.at[idx_a], rows_a, sem_a)
        cp_b = pltpu.async_copy(d1f_ref.at[idx_b], rows_b, sem_b)
        cp_a.wait()
        pltpu.sync_copy(rows_a, r12_ref.at[pl.ds(base, bpw)])
        cp_b.wait()
        pltpu.sync_copy(rows_b, r21_ref.at[pl.ds(base, bpw)])

    return pl.kernel(
        body,
        out_type=[jax.ShapeDtypeStruct((n, 128), jnp.float32)] * 2,
        mesh=mesh,
        scratch_types=[
            pltpu.VMEM((bpw,), jnp.int32),
            pltpu.VMEM((bpw,), jnp.int32),
            pltpu.VMEM((bpw, 128), jnp.float32),
            pltpu.VMEM((bpw, 128), jnp.float32),
            pltpu.SemaphoreType.DMA,
            pltpu.SemaphoreType.DMA,
        ],
    )


def kernel(distances1, distances2):
    n = distances1.shape[0]
    S = n // 128
    d = jnp.stack([distances1, distances2]).reshape(2, n, S, 128)

    prim_out = [
        jax.ShapeDtypeStruct((2, S, 128), jnp.int32),
        jax.ShapeDtypeStruct((2, S, 128), jnp.float32),
        jax.ShapeDtypeStruct((2, S, 128), jnp.int32),
    ]
    in_blk = pl.BlockSpec((1, n, S, 128), lambda i: (i, 0, 0, 0))
    out_blk = pl.BlockSpec((1, S, 128), lambda i: (i, 0, 0))
    rows, md, par = pl.pallas_call(
        _prim_one_kernel,
        grid=(2,),
        in_specs=[in_blk],
        out_specs=[out_blk] * 3,
        out_shape=prim_out,
        compiler_params=pltpu.CompilerParams(
            dimension_semantics=("parallel",)),
    )(d)

    nc, ns, lanes = _sc_info()
    sc_gather = _make_sc_gather(n, nc, ns, lanes)
    r12, r21 = sc_gather(
        distances2.reshape(n * n // 128, 128),
        distances1.reshape(n * n // 128, 128),
        rows[0].reshape(n), rows[1].reshape(n))

    fin_out = [jax.ShapeDtypeStruct((1, 1), jnp.float32)] * 4
    dist, matched, d12, d21 = pl.pallas_call(_finish_kernel, out_shape=fin_out)(
        md, par, r12, r21)
    return (dist[0, 0], matched[0, 0], d12[0, 0], d21[0, 0])


# speedup vs baseline: 3.3651x; 1.7583x over previous
import functools

import jax
import jax.numpy as jnp
from jax import lax
from jax.experimental import pallas as pl
from jax.experimental.pallas import tpu as pltpu
from jax.experimental.pallas import tpu_sc as plsc


def _prim_kernel(d1_ref, d2_ref, rows1_ref, rows2_ref, md1_ref, md2_ref,
                 matched_ref):
    n, S, L = d1_ref.shape
    iota = (lax.broadcasted_iota(jnp.int32, (S, L), 0) * L
            + lax.broadcasted_iota(jnp.int32, (S, L), 1))
    INF = jnp.float32(jnp.inf)
    root = iota == 0
    sub_iota = lax.broadcasted_iota(jnp.int32, (S, 1), 0)

    def step(d_ref, masked, md, par):
        notin = masked < INF
        lane_idx = jnp.argmin(masked, axis=1).astype(jnp.int32).reshape(S, 1)
        lane_min = jnp.min(masked, axis=1).reshape(S, 1)
        lin = sub_iota * L + lane_idx
        lm = lane_min
        for sh in (4, 2, 1):
            lm_r = jnp.roll(lm, sh, axis=0)
            lin_r = jnp.roll(lin, sh, axis=0)
            take = (lm_r < lm) | ((lm_r == lm) & (lin_r < lin))
            lm = jnp.where(take, lm_r, lm)
            lin = jnp.where(take, lin_r, lin)
        v = lin[0, 0]
        row = d_ref[v]
        is_v = iota == v
        better = (row < masked) & notin
        masked_n = jnp.where(is_v, INF, jnp.where(better, row, masked))
        upd = better & jnp.logical_not(is_v)
        par_n = jnp.where(upd, v, par)
        md_n = jnp.where(upd, row, md)
        return masked_n, md_n, par_n

    def body(i, st):
        ma1, md1, par1, ma2, md2, par2 = st
        ma1, md1, par1 = step(d1_ref, ma1, md1, par1)
        ma2, md2, par2 = step(d2_ref, ma2, md2, par2)
        ma1, md1, par1 = step(d1_ref, ma1, md1, par1)
        ma2, md2, par2 = step(d2_ref, ma2, md2, par2)
        return (ma1, md1, par1, ma2, md2, par2)

    zero_i = jnp.zeros((S, L), jnp.int32)
    row0_1 = d1_ref[0]
    row0_2 = d2_ref[0]
    init = (jnp.where(root, INF, row0_1), row0_1, zero_i,
            jnp.where(root, INF, row0_2), row0_2, zero_i)
    st = lax.fori_loop(0, (n - 2) // 2, body, init)
    ma1, md1, par1, ma2, md2, par2 = st
    _, md1, par1 = step(d1_ref, ma1, md1, par1)
    _, md2, par2 = step(d2_ref, ma2, md2, par2)

    nonroot = jnp.logical_not(root)
    matched = jnp.sum(jnp.where((par1 == par2) & nonroot, 1.0, 0.0))

    rows1_ref[...] = par1 * (n // 128) + (iota >> 7)
    rows2_ref[...] = par2 * (n // 128) + (iota >> 7)
    md1_ref[...] = md1
    md2_ref[...] = md2
    matched_ref[...] = matched.reshape(1, 1)


def _finish_kernel(md1_ref, md2_ref, r12_ref, r21_ref, dist_ref, d12_ref,
                   d21_ref):
    S, L = md1_ref.shape
    n = S * L
    iota = (lax.broadcasted_iota(jnp.int32, (S, L), 0) * L
            + lax.broadcasted_iota(jnp.int32, (S, L), 1))
    sub8 = lax.broadcasted_iota(jnp.int32, (8, 128), 0)
    lane128 = lax.broadcasted_iota(jnp.int32, (8, 128), 1)
    nonroot = iota != 0

    def diag_select(r_ref):
        y = jnp.zeros((S, L), jnp.float32)
        for a in range(n // 128):
            p = jnp.zeros((8, 128), jnp.float32)
            for k in range(16):
                t = r_ref[a * 128 + 8 * k: a * 128 + 8 * k + 8, :]
                p = p + jnp.where(lane128 == 8 * k + sub8, t, 0.0)
            d = jnp.sum(p, axis=0, keepdims=True)
            y = y + jnp.where(sub8 == a, d, 0.0)
        return y

    s12 = diag_select(r12_ref)
    s21 = diag_select(r21_ref)
    e12 = jnp.where(nonroot, md1_ref[...] - s12, 0.0)
    e21 = jnp.where(nonroot, md2_ref[...] - s21, 0.0)
    d12 = jnp.sqrt(jnp.sum(e12 * e12))
    d21 = jnp.sqrt(jnp.sum(e21 * e21))
    dist_ref[...] = (d12 + d21).reshape(1, 1)
    d12_ref[...] = d12.reshape(1, 1)
    d21_ref[...] = d21.reshape(1, 1)


def _sc_info():
    try:
        info = plsc.get_sparse_core_info()
        return info.num_cores, info.num_subcores, info.num_lanes
    except Exception:
        return 2, 16, 16


def _make_sc_gather(n, nc, ns, lanes):
    nw = nc * ns
    bpw = n // nw
    halves = bpw // lanes
    mesh = plsc.VectorSubcoreMesh(
        core_axis_name="c", subcore_axis_name="s",
        num_cores=nc, num_subcores=ns)

    def body(d2f_ref, d1f_ref, rows1_ref, rows2_ref, r12_ref, r21_ref,
             idx_a, idx_b, rows_a, rows_b, sem_a, sem_b):
        wid = lax.axis_index("s") * nc + lax.axis_index("c")
        base = wid * bpw
        pltpu.sync_copy(rows1_ref.at[pl.ds(base, bpw)], idx_a)
        pltpu.sync_copy(rows2_ref.at[pl.ds(base, bpw)], idx_b)
        cp_a = pltpu.async_copy(d2f_ref.at[idx_a], rows_a, sem_a)
        cp_b = pltpu.async_copy(d1f_ref.at[idx_b], rows_b, sem_b)
        cp_a.wait()
        pltpu.sync_copy(rows_a, r12_ref.at[pl.ds(base, bpw)])
        cp_b.wait()
        pltpu.sync_copy(rows_b, r21_ref.at[pl.ds(base, bpw)])

    return pl.kernel(
        body,
        out_type=[jax.ShapeDtypeStruct((n, 128), jnp.float32)] * 2,
        mesh=mesh,
        scratch_types=[
            pltpu.VMEM((bpw,), jnp.int32),
            pltpu.VMEM((bpw,), jnp.int32),
            pltpu.VMEM((bpw, 128), jnp.float32),
            pltpu.VMEM((bpw, 128), jnp.float32),
            pltpu.SemaphoreType.DMA,
            pltpu.SemaphoreType.DMA,
        ],
    )


def kernel(distances1, distances2):
    n = distances1.shape[0]
    S = n // 128
    d1r = distances1.reshape(n, S, 128)
    d2r = distances2.reshape(n, S, 128)

    prim_out = [
        jax.ShapeDtypeStruct((S, 128), jnp.int32),
        jax.ShapeDtypeStruct((S, 128), jnp.int32),
        jax.ShapeDtypeStruct((S, 128), jnp.float32),
        jax.ShapeDtypeStruct((S, 128), jnp.float32),
        jax.ShapeDtypeStruct((1, 1), jnp.float32),
    ]
    rows1, rows2, md1, md2, matched = pl.pallas_call(
        _prim_kernel, out_shape=prim_out)(d1r, d2r)

    nc, ns, lanes = _sc_info()
    sc_gather = _make_sc_gather(n, nc, ns, lanes)
    r12, r21 = sc_gather(
        distances2.reshape(n * n // 128, 128),
        distances1.reshape(n * n // 128, 128),
        rows1.reshape(n), rows2.reshape(n))

    fin_out = [jax.ShapeDtypeStruct((1, 1), jnp.float32)] * 3
    dist, d12, d21 = pl.pallas_call(_finish_kernel, out_shape=fin_out)(
        md1, md2, r12, r21)
    return (dist[0, 0], matched[0, 0], d12[0, 0], d21[0, 0])
